# baseline (speedup 1.0000x reference)
import jax
import jax.numpy as jnp
from jax import lax
from jax.experimental import pallas as pl
from jax.experimental.pallas import tpu as pltpu
from jax.experimental.pallas import tpu_sc as plsc

VOCAB = 100000
EMBED_DIM = 64
B = 4096
T = 200
N = B * T

NC = 2
NS = 16
NW = NC * NS

PER_W = N // NW
CHUNK = 512
STEPS = PER_W // CHUNK

TK = T * EMBED_DIM
BB = 128


def _fire_gather(table_hbm, idx_v, rows, sem, chunk_i):
    pltpu.async_copy(
        table_hbm.at[idx_v.at[pl.ds(chunk_i * CHUNK, CHUNK)]],
        rows,
        sem,
    )


def _wait_rows(table_hbm, rows, sem):
    pltpu.make_async_copy(table_hbm.at[pl.ds(0, CHUNK)], rows, sem).wait()


def _fire_out(out_hbm, rows, sem, w_base, chunk_i):
    pltpu.async_copy(rows, out_hbm.at[pl.ds(w_base + chunk_i * CHUNK, CHUNK)], sem)


def _wait_out(out_hbm, rows, sem):
    pltpu.make_async_copy(rows, out_hbm.at[pl.ds(0, CHUNK)], sem).wait()


def _body(idx_hbm, table_hbm, out_hbm, idx_v, rows0, rows1, g0, g1, o0, o1):
    wid = lax.axis_index("s") * NC + lax.axis_index("c")
    w_base = wid * PER_W
    rows = (rows0, rows1)
    gsem = (g0, g1)
    osem = (o0, o1)

    pltpu.sync_copy(idx_hbm.at[pl.ds(w_base, PER_W)], idx_v)

    _fire_gather(table_hbm, idx_v, rows[0], gsem[0], 0)
    _wait_rows(table_hbm, rows[0], gsem[0])
    _fire_out(out_hbm, rows[0], osem[0], w_base, 0)
    _fire_gather(table_hbm, idx_v, rows[1], gsem[1], 1)

    def slot(i, b):
        _wait_rows(table_hbm, rows[b], gsem[b])
        _fire_out(out_hbm, rows[b], osem[b], w_base, i)
        _wait_out(out_hbm, rows[1 - b], osem[1 - b])
        _fire_gather(table_hbm, idx_v, rows[1 - b], gsem[1 - b], i + 1)

    def pair(g, carry):
        slot(1 + 2 * g, 1)
        slot(2 + 2 * g, 0)
        return carry

    lax.fori_loop(0, (STEPS - 2) // 2, pair, 0)

    bl = (STEPS - 1) % 2
    _wait_rows(table_hbm, rows[bl], gsem[bl])
    _fire_out(out_hbm, rows[bl], osem[bl], w_base, STEPS - 1)
    _wait_out(out_hbm, rows[1 - bl], osem[1 - bl])
    _wait_out(out_hbm, rows[bl], osem[bl])


def _gather_sc(idx_flat, table):
    mesh = plsc.VectorSubcoreMesh(core_axis_name="c", subcore_axis_name="s")
    kern = pl.kernel(
        _body,
        out_type=jax.ShapeDtypeStruct((N, EMBED_DIM), jnp.float32),
        mesh=mesh,
        scratch_types=[
            pltpu.VMEM((PER_W,), jnp.int32),
            pltpu.VMEM((CHUNK, EMBED_DIM), jnp.float32),
            pltpu.VMEM((CHUNK, EMBED_DIM), jnp.float32),
            pltpu.SemaphoreType.DMA,
            pltpu.SemaphoreType.DMA,
            pltpu.SemaphoreType.DMA,
            pltpu.SemaphoreType.DMA,
        ],
        compiler_params=pltpu.CompilerParams(use_tc_tiling_on_sc=False),
    )
    return kern(idx_flat, table)


def _tbody(g_ref, o_ref):
    o_ref[...] = g_ref[...].T


def _transpose_tc(g2):
    return pl.pallas_call(
        _tbody,
        grid=(B // BB, TK // BB),
        in_specs=[pl.BlockSpec((BB, BB), lambda j, k: (j, k))],
        out_specs=pl.BlockSpec((BB, BB), lambda j, k: (k, j)),
        out_shape=jax.ShapeDtypeStruct((TK, B), jnp.float32),
    )(g2)


@jax.jit
def _embed(idx_flat, table):
    g = _gather_sc(idx_flat, table)
    g2 = g.reshape(B, TK)
    o = _transpose_tc(g2)
    o3 = o.reshape(T, EMBED_DIM, B)
    return jnp.transpose(o3, (2, 0, 1))


def kernel(action_tokens, table):
    idx_flat = action_tokens.reshape(-1).astype(jnp.int32)
    return _embed(idx_flat, table)


# speedup vs baseline: 2.8246x; 2.8246x over previous
import jax
import jax.numpy as jnp
from jax import lax
from jax.experimental import pallas as pl
from jax.experimental.pallas import tpu as pltpu
from jax.experimental.pallas import tpu_sc as plsc

VOCAB = 100000
EMBED_DIM = 64
B = 4096
T = 200
N = B * T

NC = 2
NS = 16
NW = NC * NS

PER_B = B // NW


def _fire_gather(table_hbm, idx_v, rows, sem, t):
    pltpu.async_copy(table_hbm.at[idx_v.at[t]], rows, sem)


def _wait_rows(table_hbm, rows, sem):
    pltpu.make_async_copy(table_hbm.at[pl.ds(0, PER_B)], rows, sem).wait()


def _fire_out(out_hbm, rows, sem, b0, t):
    pltpu.async_copy(rows, out_hbm.at[pl.ds(t * B + b0, PER_B)], sem)


def _wait_out(out_hbm, rows, sem):
    pltpu.make_async_copy(rows, out_hbm.at[pl.ds(0, PER_B)], sem).wait()


def _body(idxt_hbm, table_hbm, out_hbm, idx_v, rows0, rows1, g0, g1, o0, o1):
    wid = lax.axis_index("s") * NC + lax.axis_index("c")
    b0 = wid * PER_B
    rows = (rows0, rows1)
    gsem = (g0, g1)
    osem = (o0, o1)

    pltpu.sync_copy(idxt_hbm.at[:, pl.ds(b0, PER_B)], idx_v)

    _fire_gather(table_hbm, idx_v, rows[0], gsem[0], 0)
    _wait_rows(table_hbm, rows[0], gsem[0])
    _fire_out(out_hbm, rows[0], osem[0], b0, 0)
    _fire_gather(table_hbm, idx_v, rows[1], gsem[1], 1)

    def slot(t, b):
        _wait_rows(table_hbm, rows[b], gsem[b])
        _fire_out(out_hbm, rows[b], osem[b], b0, t)
        _wait_out(out_hbm, rows[1 - b], osem[1 - b])
        _fire_gather(table_hbm, idx_v, rows[1 - b], gsem[1 - b], t + 1)

    def pair(g, carry):
        slot(1 + 2 * g, 1)
        slot(2 + 2 * g, 0)
        return carry

    lax.fori_loop(0, (T - 2) // 2, pair, 0)

    bl = (T - 1) % 2
    _wait_rows(table_hbm, rows[bl], gsem[bl])
    _fire_out(out_hbm, rows[bl], osem[bl], b0, T - 1)
    _wait_out(out_hbm, rows[1 - bl], osem[1 - bl])
    _wait_out(out_hbm, rows[bl], osem[bl])


def _gather_sc(idx_t, table):
    mesh = plsc.VectorSubcoreMesh(core_axis_name="c", subcore_axis_name="s")
    kern = pl.kernel(
        _body,
        out_type=jax.ShapeDtypeStruct((N, EMBED_DIM), jnp.float32),
        mesh=mesh,
        scratch_types=[
            pltpu.VMEM((T, PER_B), jnp.int32),
            pltpu.VMEM((PER_B, EMBED_DIM), jnp.float32),
            pltpu.VMEM((PER_B, EMBED_DIM), jnp.float32),
            pltpu.SemaphoreType.DMA,
            pltpu.SemaphoreType.DMA,
            pltpu.SemaphoreType.DMA,
            pltpu.SemaphoreType.DMA,
        ],
        compiler_params=pltpu.CompilerParams(use_tc_tiling_on_sc=False),
    )
    return kern(idx_t, table)


@jax.jit
def _embed(idx_t, table):
    g = _gather_sc(idx_t, table)
    g3 = g.reshape(T, B, EMBED_DIM)
    return jnp.transpose(g3, (1, 0, 2))


def kernel(action_tokens, table):
    idx_t = action_tokens.T.astype(jnp.int32)
    return _embed(idx_t, table)
